# baseline (speedup 1.0000x reference)
import functools

import jax
import jax.numpy as jnp
from jax import lax
from jax.experimental import pallas as pl
from jax.experimental.pallas import tpu as pltpu
from jax.experimental.pallas import tpu_sc as plsc

CUTOFF = 0.5
E_PARENT = 6400000
E_FILTER = 1600000

NW = 32
PER_W = E_FILTER // NW
CHUNK = 10000
N_CHUNKS = PER_W // CHUNK

_PI = 3.14159265358979
_C3 = -1.0 / 6.0
_C5 = 1.0 / 120.0
_C7 = -1.0 / 5040.0
_C9 = 1.0 / 362880.0


@functools.partial(
    pl.kernel,
    out_type=[
        jax.ShapeDtypeStruct((E_FILTER * 3,), jnp.float32),
        jax.ShapeDtypeStruct((E_FILTER,), jnp.float32),
        jax.ShapeDtypeStruct((E_FILTER,), jnp.float32),
        jax.ShapeDtypeStruct((E_FILTER,), jnp.int32),
    ],
    mesh=plsc.VectorSubcoreMesh(core_axis_name="c", subcore_axis_name="s"),
    scratch_types=[
        pltpu.VMEM((CHUNK,), jnp.int32),
        pltpu.VMEM((3 * CHUNK,), jnp.int32),
        pltpu.VMEM((3 * CHUNK,), jnp.float32),
        pltpu.VMEM((CHUNK,), jnp.float32),
        pltpu.VMEM((CHUNK,), jnp.float32),
        pltpu.VMEM((CHUNK,), jnp.int32),
        pltpu.SemaphoreType.DMA,
        pltpu.SemaphoreType.DMA,
    ],
)
def _gather_switch(vec_hbm, dist_hbm, idx_hbm,
                   v_out, d_out, sw_out, m_out,
                   idx_v, idxf_v, v_v, d_v, sw_v, m_v, sem_v, sem_d):
    wid = lax.axis_index("s") * 2 + lax.axis_index("c")
    base0 = wid * PER_W

    def chunk_body(j, carry):
        base = base0 + j * CHUNK
        pltpu.sync_copy(idx_hbm.at[pl.ds(base, CHUNK)], idx_v)

        def expand(m, carry2):
            a = idx_v[pl.ds(m * 16, 16)]
            lane = lax.iota(jnp.int32, 16)
            for c in range(3):
                k = lane + (16 * c)
                q = (k * 21846) >> 16
                r = k - q * 3
                g = a[q]
                idxf_v[pl.ds(m * 48 + c * 16, 16)] = g + g + g + r
            return carry2

        lax.fori_loop(0, CHUNK // 16, expand, 0, unroll=4)

        cp_v = pltpu.async_copy(vec_hbm.at[idxf_v], v_v, sem_v)
        cp_d = pltpu.async_copy(dist_hbm.at[idx_v], d_v, sem_d)
        cp_d.wait()

        def ew(i, carry2):
            d16 = d_v[pl.ds(i * 16, 16)]
            s = (d16 * (2.0 * _PI)) - (0.5 * _PI)
            s2 = s * s
            p = s2 * _C9 + _C7
            p = s2 * p + _C5
            p = s2 * p + _C3
            sin_s = s + s * (s2 * p)
            mask = d16 < CUTOFF
            sw_v[pl.ds(i * 16, 16)] = jnp.where(mask, 0.5 - 0.5 * sin_s,
                                                jnp.zeros((16,), jnp.float32))
            m_v[pl.ds(i * 16, 16)] = jnp.where(mask, jnp.ones((16,), jnp.int32),
                                               jnp.zeros((16,), jnp.int32))
            return carry2

        lax.fori_loop(0, CHUNK // 16, ew, 0, unroll=4)
        cp_v.wait()
        pltpu.sync_copy(v_v, v_out.at[pl.ds(base * 3, CHUNK * 3)])
        pltpu.sync_copy(d_v, d_out.at[pl.ds(base, CHUNK)])
        pltpu.sync_copy(sw_v, sw_out.at[pl.ds(base, CHUNK)])
        pltpu.sync_copy(m_v, m_out.at[pl.ds(base, CHUNK)])
        return carry

    lax.fori_loop(0, N_CHUNKS, chunk_body, 0)


def kernel(vec, distances, filter_indices):
    vflat, d, sw, m = _gather_switch(vec.reshape(-1), distances, filter_indices)
    return vflat.reshape(E_FILTER, 3), d, sw, m.astype(jnp.bool_)


# speedup vs baseline: 4.5923x; 4.5923x over previous
import functools

import jax
import jax.numpy as jnp
from jax import lax
from jax.experimental import pallas as pl
from jax.experimental.pallas import tpu as pltpu
from jax.experimental.pallas import tpu_sc as plsc

CUTOFF = 0.5
E_PARENT = 6400000
E_FILTER = 1600000

NW = 32
PER_W = E_FILTER // NW
CHUNK = 10000
N_CHUNKS = PER_W // CHUNK

_PI = 3.14159265358979
_C3 = -1.0 / 6.0
_C5 = 1.0 / 120.0
_C7 = -1.0 / 5040.0
_C9 = 1.0 / 362880.0


@functools.partial(
    pl.kernel,
    out_type=[
        jax.ShapeDtypeStruct((3 * E_FILTER,), jnp.float32),
        jax.ShapeDtypeStruct((E_FILTER,), jnp.float32),
        jax.ShapeDtypeStruct((E_FILTER,), jnp.float32),
        jax.ShapeDtypeStruct((E_FILTER,), jnp.int32),
    ],
    mesh=plsc.VectorSubcoreMesh(core_axis_name="c", subcore_axis_name="s"),
    scratch_types=[
        pltpu.VMEM((CHUNK,), jnp.int32),
        pltpu.VMEM((CHUNK,), jnp.float32),
        pltpu.VMEM((CHUNK,), jnp.float32),
        pltpu.VMEM((CHUNK,), jnp.float32),
        pltpu.VMEM((CHUNK,), jnp.float32),
        pltpu.VMEM((CHUNK,), jnp.float32),
        pltpu.VMEM((CHUNK,), jnp.int32),
        pltpu.SemaphoreType.DMA,
        pltpu.SemaphoreType.DMA,
    ],
)
def _gather_switch(vecp_hbm, dist_hbm, idx_hbm,
                   v_out, d_out, sw_out, m_out,
                   idx_v, v0_v, v1_v, v2_v, d_v, sw_v, m_v, sem_v, sem_d):
    wid = lax.axis_index("s") * 2 + lax.axis_index("c")
    base0 = wid * PER_W

    def chunk_body(j, carry):
        base = base0 + j * CHUNK
        pltpu.sync_copy(idx_hbm.at[pl.ds(base, CHUNK)], idx_v)

        cps = []
        for c, vc_v in enumerate((v0_v, v1_v, v2_v)):
            cps.append(pltpu.async_copy(
                vecp_hbm.at[pl.ds(c * E_PARENT, E_PARENT)].at[idx_v],
                vc_v, sem_v))
        cp_d = pltpu.async_copy(dist_hbm.at[idx_v], d_v, sem_d)
        cp_d.wait()

        def ew(i, carry2):
            d16 = d_v[pl.ds(i * 16, 16)]
            s = (d16 * (2.0 * _PI)) - (0.5 * _PI)
            s2 = s * s
            p = s2 * _C9 + _C7
            p = s2 * p + _C5
            p = s2 * p + _C3
            sin_s = s + s * (s2 * p)
            mask = d16 < CUTOFF
            sw_v[pl.ds(i * 16, 16)] = jnp.where(mask, 0.5 - 0.5 * sin_s,
                                                jnp.zeros((16,), jnp.float32))
            m_v[pl.ds(i * 16, 16)] = jnp.where(mask, jnp.ones((16,), jnp.int32),
                                               jnp.zeros((16,), jnp.int32))
            return carry2

        lax.fori_loop(0, CHUNK // 16, ew, 0, unroll=4)
        for cp in cps:
            cp.wait()
        for c, vc_v in enumerate((v0_v, v1_v, v2_v)):
            pltpu.sync_copy(vc_v, v_out.at[pl.ds(c * E_FILTER + base, CHUNK)])
        pltpu.sync_copy(d_v, d_out.at[pl.ds(base, CHUNK)])
        pltpu.sync_copy(sw_v, sw_out.at[pl.ds(base, CHUNK)])
        pltpu.sync_copy(m_v, m_out.at[pl.ds(base, CHUNK)])
        return carry

    lax.fori_loop(0, N_CHUNKS, chunk_body, 0)


def kernel(vec, distances, filter_indices):
    vec_planar = vec.T.reshape(-1)
    vp, d, sw, m = _gather_switch(vec_planar, distances, filter_indices)
    return vp.reshape(3, E_FILTER).T, d, sw, m.astype(jnp.bool_)


# speedup vs baseline: 22.4485x; 4.8883x over previous
import functools

import jax
import jax.numpy as jnp
from jax import lax
from jax.experimental import pallas as pl
from jax.experimental.pallas import tpu as pltpu
from jax.experimental.pallas import tpu_sc as plsc

CUTOFF = 0.5
E_PARENT = 6400000
E_FILTER = 1600000

NW = 32
NBLK = E_FILTER // 128
BASE_BLKS = NBLK // NW
EXTRA = NBLK - BASE_BLKS * NW
CB = 78
CN = CB * 128
N_CHUNKS = BASE_BLKS // CB

_PI = 3.14159265358979
_C3 = -1.0 / 6.0
_C5 = 1.0 / 120.0
_C7 = -1.0 / 5040.0
_C9 = 1.0 / 362880.0


def _expand_step(idx_v, idxf_v, g):
    a = idx_v[pl.ds(g * 16, 16)]
    p0 = (a >> 7) * 384 + (a & 127)
    b = g >> 3
    r = g & 7
    dst = b * 384 + r * 16
    idxf_v[pl.ds(dst, 16)] = p0
    idxf_v[pl.ds(dst + 128, 16)] = p0 + 128
    idxf_v[pl.ds(dst + 256, 16)] = p0 + 256


def _switch_step(d_v, sw_v, m_v, i):
    d16 = d_v[pl.ds(i * 16, 16)]
    s = (d16 * (2.0 * _PI)) - (0.5 * _PI)
    s2 = s * s
    p = s2 * _C9 + _C7
    p = s2 * p + _C5
    p = s2 * p + _C3
    sin_s = s + s * (s2 * p)
    mask = d16 < CUTOFF
    sw_v[pl.ds(i * 16, 16)] = jnp.where(mask, 0.5 - 0.5 * sin_s,
                                        jnp.zeros((16,), jnp.float32))
    m_v[pl.ds(i * 16, 16)] = jnp.where(mask, jnp.ones((16,), jnp.int32),
                                       jnp.zeros((16,), jnp.int32))


@functools.partial(
    pl.kernel,
    out_type=[
        jax.ShapeDtypeStruct((3 * E_FILTER,), jnp.float32),
        jax.ShapeDtypeStruct((E_FILTER,), jnp.float32),
        jax.ShapeDtypeStruct((E_FILTER,), jnp.float32),
        jax.ShapeDtypeStruct((E_FILTER,), jnp.int32),
    ],
    mesh=plsc.VectorSubcoreMesh(core_axis_name="c", subcore_axis_name="s"),
    scratch_types=[
        pltpu.VMEM((CN,), jnp.int32),
        pltpu.VMEM((3 * CN,), jnp.int32),
        pltpu.VMEM((3 * CN,), jnp.float32),
        pltpu.VMEM((CN,), jnp.float32),
        pltpu.VMEM((CN,), jnp.float32),
        pltpu.VMEM((CN,), jnp.int32),
        pltpu.VMEM((128,), jnp.int32),
        pltpu.VMEM((384,), jnp.int32),
        pltpu.VMEM((384,), jnp.float32),
        pltpu.VMEM((128,), jnp.float32),
        pltpu.VMEM((128,), jnp.float32),
        pltpu.VMEM((128,), jnp.int32),
        pltpu.SemaphoreType.DMA,
        pltpu.SemaphoreType.DMA,
    ],
)
def _gather_switch(vpf_hbm, dist_hbm, idx_hbm,
                   v_out, d_out, sw_out, m_out,
                   idx_v, idxf_v, v_v, d_v, sw_v, m_v,
                   idx2_v, idxf2_v, v2_v, d2_v, sw2_v, m2_v,
                   sem_v, sem_d):
    wid = lax.axis_index("s") * 2 + lax.axis_index("c")
    start_blk = wid * BASE_BLKS + jnp.minimum(wid, EXTRA)

    def chunk_body(j, carry):
        base_i = (start_blk + j * CB) * 128
        pltpu.sync_copy(idx_hbm.at[pl.ds(base_i, CN)], idx_v)

        def expand(g, carry2):
            _expand_step(idx_v, idxf_v, g)
            return carry2

        lax.fori_loop(0, CN // 16, expand, 0, unroll=4)

        cp_v = pltpu.async_copy(vpf_hbm.at[idxf_v], v_v, sem_v)
        cp_d = pltpu.async_copy(dist_hbm.at[idx_v], d_v, sem_d)
        cp_d.wait()

        def ew(i, carry2):
            _switch_step(d_v, sw_v, m_v, i)
            return carry2

        lax.fori_loop(0, CN // 16, ew, 0, unroll=4)
        cp_v.wait()
        pltpu.sync_copy(v_v, v_out.at[pl.ds(base_i * 3, 3 * CN)])
        pltpu.sync_copy(d_v, d_out.at[pl.ds(base_i, CN)])
        pltpu.sync_copy(sw_v, sw_out.at[pl.ds(base_i, CN)])
        pltpu.sync_copy(m_v, m_out.at[pl.ds(base_i, CN)])
        return carry

    lax.fori_loop(0, N_CHUNKS, chunk_body, 0)

    @pl.when(wid < EXTRA)
    def _():
        base_i = (start_blk + BASE_BLKS) * 128
        pltpu.sync_copy(idx_hbm.at[pl.ds(base_i, 128)], idx2_v)

        def expand2(g, carry2):
            _expand_step(idx2_v, idxf2_v, g)
            return carry2

        lax.fori_loop(0, 8, expand2, 0, unroll=4)

        cp_v = pltpu.async_copy(vpf_hbm.at[idxf2_v], v2_v, sem_v)
        cp_d = pltpu.async_copy(dist_hbm.at[idx2_v], d2_v, sem_d)
        cp_d.wait()

        def ew2(i, carry2):
            _switch_step(d2_v, sw2_v, m2_v, i)
            return carry2

        lax.fori_loop(0, 8, ew2, 0, unroll=4)
        cp_v.wait()
        pltpu.sync_copy(v2_v, v_out.at[pl.ds(base_i * 3, 384)])
        pltpu.sync_copy(d2_v, d_out.at[pl.ds(base_i, 128)])
        pltpu.sync_copy(sw2_v, sw_out.at[pl.ds(base_i, 128)])
        pltpu.sync_copy(m2_v, m_out.at[pl.ds(base_i, 128)])


def kernel(vec, distances, filter_indices):
    vpf = vec.T.reshape(3, E_PARENT // 128, 128).transpose(1, 0, 2).reshape(-1)
    vflat, d, sw, m = _gather_switch(vpf, distances, filter_indices)
    v = vflat.reshape(E_FILTER // 128, 3, 128).transpose(1, 0, 2)
    v = v.reshape(3, E_FILTER).T
    return v, d, sw, m.astype(jnp.bool_)


# speedup vs baseline: 24.4329x; 1.0884x over previous
import functools

import jax
import jax.numpy as jnp
from jax import lax
from jax.experimental import pallas as pl
from jax.experimental.pallas import tpu as pltpu
from jax.experimental.pallas import tpu_sc as plsc

CUTOFF = 0.5
E_PARENT = 6400000
E_FILTER = 1600000

NW = 32
NBLK = E_FILTER // 128
BASE_BLKS = NBLK // NW
EXTRA = NBLK - BASE_BLKS * NW

CB_A = 78
CN_A = CB_A * 128
NCH_A = BASE_BLKS // CB_A

CB_B = 65
CN_B = CB_B * 128
NCH_B = BASE_BLKS // CB_B

_PI = 3.14159265358979
_C3 = -1.0 / 6.0
_C5 = 1.0 / 120.0
_C7 = -1.0 / 5040.0
_C9 = 1.0 / 362880.0


def _expand_step(idx_v, idxf_v, g):
    a = idx_v[pl.ds(g * 16, 16)]
    p0 = (a >> 7) * 384 + (a & 127)
    b = g >> 3
    r = g & 7
    dst = b * 384 + r * 16
    idxf_v[pl.ds(dst, 16)] = p0
    idxf_v[pl.ds(dst + 128, 16)] = p0 + 128
    idxf_v[pl.ds(dst + 256, 16)] = p0 + 256


def _switch_step(d_v, sw_v, m_v, i):
    d16 = d_v[pl.ds(i * 16, 16)]
    s = (d16 * (2.0 * _PI)) - (0.5 * _PI)
    s2 = s * s
    p = s2 * _C9 + _C7
    p = s2 * p + _C5
    p = s2 * p + _C3
    sin_s = s + s * (s2 * p)
    mask = d16 < CUTOFF
    sw_v[pl.ds(i * 16, 16)] = jnp.where(mask, 0.5 - 0.5 * sin_s,
                                        jnp.zeros((16,), jnp.float32))
    m_v[pl.ds(i * 16, 16)] = jnp.where(mask, jnp.ones((16,), jnp.int32),
                                       jnp.zeros((16,), jnp.int32))


def _worker_start(_=None):
    wid = lax.axis_index("s") * 2 + lax.axis_index("c")
    return wid, wid * BASE_BLKS + jnp.minimum(wid, EXTRA)


@functools.partial(
    pl.kernel,
    out_type=[
        jax.ShapeDtypeStruct((E_FILTER,), jnp.float32),
        jax.ShapeDtypeStruct((E_FILTER,), jnp.float32),
        jax.ShapeDtypeStruct((E_FILTER,), jnp.int32),
    ],
    mesh=plsc.VectorSubcoreMesh(core_axis_name="c", subcore_axis_name="s"),
    scratch_types=[
        pltpu.VMEM((CN_A,), jnp.int32),
        pltpu.VMEM((CN_A,), jnp.float32),
        pltpu.VMEM((CN_A,), jnp.float32),
        pltpu.VMEM((CN_A,), jnp.int32),
        pltpu.SemaphoreType.DMA,
    ],
)
def _dist_switch(dist_hbm, idx_hbm, d_out, sw_out, m_out,
                 idx_v, d_v, sw_v, m_v, sem_d):
    wid, start_blk = _worker_start()

    def chunk_body(j, carry):
        base_i = (start_blk + j * CB_A) * 128
        pltpu.sync_copy(idx_hbm.at[pl.ds(base_i, CN_A)], idx_v)
        pltpu.async_copy(dist_hbm.at[idx_v], d_v, sem_d).wait()

        def ew(i, carry2):
            _switch_step(d_v, sw_v, m_v, i)
            return carry2

        lax.fori_loop(0, CN_A // 16, ew, 0, unroll=4)
        pltpu.sync_copy(d_v, d_out.at[pl.ds(base_i, CN_A)])
        pltpu.sync_copy(sw_v, sw_out.at[pl.ds(base_i, CN_A)])
        pltpu.sync_copy(m_v, m_out.at[pl.ds(base_i, CN_A)])
        return carry

    lax.fori_loop(0, NCH_A, chunk_body, 0)

    @pl.when(wid < EXTRA)
    def _():
        base_i = (start_blk + BASE_BLKS) * 128
        pltpu.sync_copy(idx_hbm.at[pl.ds(base_i, 128)], idx_v.at[pl.ds(0, 128)])
        pltpu.async_copy(dist_hbm.at[idx_v.at[pl.ds(0, 128)]],
                         d_v.at[pl.ds(0, 128)], sem_d).wait()

        def ew2(i, carry2):
            _switch_step(d_v, sw_v, m_v, i)
            return carry2

        lax.fori_loop(0, 8, ew2, 0, unroll=4)
        pltpu.sync_copy(d_v.at[pl.ds(0, 128)], d_out.at[pl.ds(base_i, 128)])
        pltpu.sync_copy(sw_v.at[pl.ds(0, 128)], sw_out.at[pl.ds(base_i, 128)])
        pltpu.sync_copy(m_v.at[pl.ds(0, 128)], m_out.at[pl.ds(base_i, 128)])


@functools.partial(
    pl.kernel,
    out_type=jax.ShapeDtypeStruct((3 * E_FILTER,), jnp.float32),
    mesh=plsc.VectorSubcoreMesh(core_axis_name="c", subcore_axis_name="s"),
    scratch_types=[
        pltpu.VMEM((CN_B,), jnp.int32),
        pltpu.VMEM((3 * CN_B,), jnp.int32),
        pltpu.VMEM((3 * CN_B,), jnp.int32),
        pltpu.VMEM((3 * CN_B,), jnp.float32),
        pltpu.VMEM((3 * CN_B,), jnp.float32),
        pltpu.SemaphoreType.DMA,
        pltpu.SemaphoreType.DMA,
    ],
)
def _vec_gather(vpf_hbm, idx_hbm, v_out,
                idx_v, idxf0, idxf1, v0, v1, sem0, sem1):
    wid, start_blk = _worker_start()
    idxfs = (idxf0, idxf1)
    vs = (v0, v1)
    sems = (sem0, sem1)

    def stage_expand(j, idxf_v):
        base_i = (start_blk + j * CB_B) * 128
        pltpu.sync_copy(idx_hbm.at[pl.ds(base_i, CN_B)], idx_v)

        def expand(g, carry2):
            _expand_step(idx_v, idxf_v, g)
            return carry2

        lax.fori_loop(0, CN_B // 16, expand, 0, unroll=4)

    stage_expand(0, idxf0)
    cps = [pltpu.async_copy(vpf_hbm.at[idxf0], v0, sem0), None]
    for j in range(NCH_B):
        p = j % 2
        q = (j + 1) % 2
        if j + 1 < NCH_B:
            stage_expand(j + 1, idxfs[q])
            cps[q] = pltpu.async_copy(vpf_hbm.at[idxfs[q]], vs[q], sems[q])
        else:
            @pl.when(wid < EXTRA)
            def _():
                base_i = (start_blk + BASE_BLKS) * 128
                pltpu.sync_copy(idx_hbm.at[pl.ds(base_i, 128)],
                                idx_v.at[pl.ds(0, 128)])

                def expand2(g, carry2):
                    _expand_step(idx_v, idxfs[q], g)
                    return carry2

                lax.fori_loop(0, 8, expand2, 0, unroll=4)
                pltpu.async_copy(vpf_hbm.at[idxfs[q].at[pl.ds(0, 384)]],
                                 vs[q].at[pl.ds(0, 384)], sems[q]).wait()
                pltpu.sync_copy(vs[q].at[pl.ds(0, 384)],
                                v_out.at[pl.ds(base_i * 3, 384)])
        base_i = (start_blk + j * CB_B) * 128
        cps[p].wait()
        pltpu.sync_copy(vs[p], v_out.at[pl.ds(base_i * 3, 3 * CN_B)])


def kernel(vec, distances, filter_indices):
    d, sw, m = _dist_switch(distances, filter_indices)
    vpf = vec.T.reshape(3, E_PARENT // 128, 128).transpose(1, 0, 2).reshape(-1)
    vflat = _vec_gather(vpf, filter_indices)
    v = vflat.reshape(E_FILTER // 128, 3, 128).transpose(1, 0, 2)
    v = v.reshape(3, E_FILTER).T
    return v, d, sw, m.astype(jnp.bool_)


# speedup vs baseline: 25.5708x; 1.0466x over previous
import functools

import jax
import jax.numpy as jnp
from jax import lax
from jax.experimental import pallas as pl
from jax.experimental.pallas import tpu as pltpu
from jax.experimental.pallas import tpu_sc as plsc

CUTOFF = 0.5
E_PARENT = 6400000
E_FILTER = 1600000

NW = 32
NBLK = E_FILTER // 128
BASE_BLKS = NBLK // NW
EXTRA = NBLK - BASE_BLKS * NW

CB_A = 78
CN_A = CB_A * 128
NCH_A = BASE_BLKS // CB_A

CB_B = 65
CN_B = CB_B * 128
NCH_B = BASE_BLKS // CB_B

_PI = 3.14159265358979
_C3 = -1.0 / 6.0
_C5 = 1.0 / 120.0
_C7 = -1.0 / 5040.0
_C9 = 1.0 / 362880.0


def _expand_step(idx_v, idxf_v, g):
    a = idx_v[pl.ds(g * 16, 16)]
    p0 = (a >> 7) * 384 + (a & 127)
    b = g >> 3
    r = g & 7
    dst = b * 384 + r * 16
    idxf_v[pl.ds(dst, 16)] = p0
    idxf_v[pl.ds(dst + 128, 16)] = p0 + 128
    idxf_v[pl.ds(dst + 256, 16)] = p0 + 256


def _switch_step(d_v, sw_v, m_v, i):
    d16 = d_v[pl.ds(i * 16, 16)]
    s = (d16 * (2.0 * _PI)) - (0.5 * _PI)
    s2 = s * s
    p = s2 * _C9 + _C7
    p = s2 * p + _C5
    p = s2 * p + _C3
    sin_s = s + s * (s2 * p)
    mask = d16 < CUTOFF
    sw_v[pl.ds(i * 16, 16)] = jnp.where(mask, 0.5 - 0.5 * sin_s,
                                        jnp.zeros((16,), jnp.float32))
    m_v[pl.ds(i * 16, 16)] = jnp.where(mask, jnp.ones((16,), jnp.int32),
                                       jnp.zeros((16,), jnp.int32))


def _worker_start(_=None):
    wid = lax.axis_index("s") * 2 + lax.axis_index("c")
    return wid, wid * BASE_BLKS + jnp.minimum(wid, EXTRA)


@functools.partial(
    pl.kernel,
    out_type=[
        jax.ShapeDtypeStruct((E_FILTER,), jnp.float32),
        jax.ShapeDtypeStruct((E_FILTER,), jnp.float32),
        jax.ShapeDtypeStruct((E_FILTER,), jnp.int32),
    ],
    mesh=plsc.VectorSubcoreMesh(core_axis_name="c", subcore_axis_name="s"),
    scratch_types=[
        pltpu.VMEM((CN_A,), jnp.int32),
        pltpu.VMEM((CN_A,), jnp.int32),
        pltpu.VMEM((CN_A,), jnp.float32),
        pltpu.VMEM((CN_A,), jnp.float32),
        pltpu.VMEM((CN_A,), jnp.float32),
        pltpu.VMEM((CN_A,), jnp.int32),
        pltpu.SemaphoreType.DMA,
        pltpu.SemaphoreType.DMA,
    ],
)
def _dist_switch(dist_hbm, idx_hbm, d_out, sw_out, m_out,
                 idx0, idx1, d0, d1, sw_v, m_v, sem0, sem1):
    wid, start_blk = _worker_start()
    idxs = (idx0, idx1)
    ds = (d0, d1)
    sems = (sem0, sem1)

    def stage_fire(j):
        p = j % 2
        base_i = (start_blk + j * CB_A) * 128
        pltpu.sync_copy(idx_hbm.at[pl.ds(base_i, CN_A)], idxs[p])
        return pltpu.async_copy(dist_hbm.at[idxs[p]], ds[p], sems[p])

    cps = [stage_fire(0), None]
    for j in range(NCH_A):
        p = j % 2
        q = (j + 1) % 2
        if j + 1 < NCH_A:
            cps[q] = stage_fire(j + 1)
        else:
            @pl.when(wid < EXTRA)
            def _():
                base_i = (start_blk + BASE_BLKS) * 128
                pltpu.sync_copy(idx_hbm.at[pl.ds(base_i, 128)],
                                idxs[q].at[pl.ds(0, 128)])
                pltpu.async_copy(dist_hbm.at[idxs[q].at[pl.ds(0, 128)]],
                                 ds[q].at[pl.ds(0, 128)], sems[q]).wait()

                def ew2(i, carry2):
                    _switch_step(ds[q], sw_v, m_v, i)
                    return carry2

                lax.fori_loop(0, 8, ew2, 0, unroll=4)
                pltpu.sync_copy(ds[q].at[pl.ds(0, 128)],
                                d_out.at[pl.ds(base_i, 128)])
                pltpu.sync_copy(sw_v.at[pl.ds(0, 128)],
                                sw_out.at[pl.ds(base_i, 128)])
                pltpu.sync_copy(m_v.at[pl.ds(0, 128)],
                                m_out.at[pl.ds(base_i, 128)])
        base_i = (start_blk + j * CB_A) * 128
        cps[p].wait()

        def ew(i, carry2):
            _switch_step(ds[p], sw_v, m_v, i)
            return carry2

        lax.fori_loop(0, CN_A // 16, ew, 0, unroll=4)
        pltpu.sync_copy(ds[p], d_out.at[pl.ds(base_i, CN_A)])
        pltpu.sync_copy(sw_v, sw_out.at[pl.ds(base_i, CN_A)])
        pltpu.sync_copy(m_v, m_out.at[pl.ds(base_i, CN_A)])


@functools.partial(
    pl.kernel,
    out_type=jax.ShapeDtypeStruct((3 * E_FILTER,), jnp.float32),
    mesh=plsc.VectorSubcoreMesh(core_axis_name="c", subcore_axis_name="s"),
    scratch_types=[
        pltpu.VMEM((CN_B,), jnp.int32),
        pltpu.VMEM((3 * CN_B,), jnp.int32),
        pltpu.VMEM((3 * CN_B,), jnp.int32),
        pltpu.VMEM((3 * CN_B,), jnp.float32),
        pltpu.VMEM((3 * CN_B,), jnp.float32),
        pltpu.SemaphoreType.DMA,
        pltpu.SemaphoreType.DMA,
    ],
)
def _vec_gather(vpf_hbm, idx_hbm, v_out,
                idx_v, idxf0, idxf1, v0, v1, sem0, sem1):
    wid, start_blk = _worker_start()
    idxfs = (idxf0, idxf1)
    vs = (v0, v1)
    sems = (sem0, sem1)

    def stage_expand(j, idxf_v):
        base_i = (start_blk + j * CB_B) * 128
        pltpu.sync_copy(idx_hbm.at[pl.ds(base_i, CN_B)], idx_v)

        def expand(g, carry2):
            _expand_step(idx_v, idxf_v, g)
            return carry2

        lax.fori_loop(0, CN_B // 16, expand, 0, unroll=4)

    stage_expand(0, idxf0)
    cps = [pltpu.async_copy(vpf_hbm.at[idxf0], v0, sem0), None]
    for j in range(NCH_B):
        p = j % 2
        q = (j + 1) % 2
        if j + 1 < NCH_B:
            stage_expand(j + 1, idxfs[q])
            cps[q] = pltpu.async_copy(vpf_hbm.at[idxfs[q]], vs[q], sems[q])
        else:
            @pl.when(wid < EXTRA)
            def _():
                base_i = (start_blk + BASE_BLKS) * 128
                pltpu.sync_copy(idx_hbm.at[pl.ds(base_i, 128)],
                                idx_v.at[pl.ds(0, 128)])

                def expand2(g, carry2):
                    _expand_step(idx_v, idxfs[q], g)
                    return carry2

                lax.fori_loop(0, 8, expand2, 0, unroll=4)
                pltpu.async_copy(vpf_hbm.at[idxfs[q].at[pl.ds(0, 384)]],
                                 vs[q].at[pl.ds(0, 384)], sems[q]).wait()
                pltpu.sync_copy(vs[q].at[pl.ds(0, 384)],
                                v_out.at[pl.ds(base_i * 3, 384)])
        base_i = (start_blk + j * CB_B) * 128
        cps[p].wait()
        pltpu.sync_copy(vs[p], v_out.at[pl.ds(base_i * 3, 3 * CN_B)])


def kernel(vec, distances, filter_indices):
    d, sw, m = _dist_switch(distances, filter_indices)
    vpf = vec.T.reshape(3, E_PARENT // 128, 128).transpose(1, 0, 2).reshape(-1)
    vflat = _vec_gather(vpf, filter_indices)
    v = vflat.reshape(E_FILTER // 128, 3, 128).transpose(1, 0, 2)
    v = v.reshape(3, E_FILTER).T
    return v, d, sw, m.astype(jnp.bool_)


# speedup vs baseline: 25.6284x; 1.0023x over previous
import functools

import jax
import jax.numpy as jnp
from jax import lax
from jax.experimental import pallas as pl
from jax.experimental.pallas import tpu as pltpu
from jax.experimental.pallas import tpu_sc as plsc

CUTOFF = 0.5
E_PARENT = 6400000
E_FILTER = 1600000

NW = 32
NBLK = E_FILTER // 128
BASE_BLKS = NBLK // NW
EXTRA = NBLK - BASE_BLKS * NW

CB_A = 78
CN_A = CB_A * 128
NCH_A = BASE_BLKS // CB_A

CB_B = 65
CN_B = CB_B * 128
NCH_B = BASE_BLKS // CB_B

_PI = 3.14159265358979
_C3 = -1.0 / 6.0
_C5 = 1.0 / 120.0
_C7 = -1.0 / 5040.0
_C9 = 1.0 / 362880.0


def _expand_step(idx_v, idxf_v, g):
    a = idx_v[pl.ds(g * 16, 16)]
    p0 = (a >> 7) * 384 + (a & 127)
    b = g >> 3
    r = g & 7
    dst = b * 384 + r * 16
    idxf_v[pl.ds(dst, 16)] = p0
    idxf_v[pl.ds(dst + 128, 16)] = p0 + 128
    idxf_v[pl.ds(dst + 256, 16)] = p0 + 256


def _switch_step(d_v, sw_v, m_v, i):
    d16 = d_v[pl.ds(i * 16, 16)]
    s = (d16 * (2.0 * _PI)) - (0.5 * _PI)
    s2 = s * s
    p = s2 * _C9 + _C7
    p = s2 * p + _C5
    p = s2 * p + _C3
    sin_s = s + s * (s2 * p)
    mask = d16 < CUTOFF
    sw_v[pl.ds(i * 16, 16)] = jnp.where(mask, 0.5 - 0.5 * sin_s,
                                        jnp.zeros((16,), jnp.float32))
    m_v[pl.ds(i * 16, 16)] = jnp.where(mask, jnp.ones((16,), jnp.int32),
                                       jnp.zeros((16,), jnp.int32))


def _worker_start(_=None):
    wid = lax.axis_index("s") * 2 + lax.axis_index("c")
    return wid, wid * BASE_BLKS + jnp.minimum(wid, EXTRA)


@functools.partial(
    pl.kernel,
    out_type=[
        jax.ShapeDtypeStruct((E_FILTER,), jnp.float32),
        jax.ShapeDtypeStruct((E_FILTER,), jnp.float32),
        jax.ShapeDtypeStruct((E_FILTER,), jnp.int32),
    ],
    mesh=plsc.VectorSubcoreMesh(core_axis_name="c", subcore_axis_name="s"),
    scratch_types=[
        pltpu.VMEM((CN_A,), jnp.int32),
        pltpu.VMEM((CN_A,), jnp.int32),
        pltpu.VMEM((CN_A,), jnp.float32),
        pltpu.VMEM((CN_A,), jnp.float32),
        pltpu.VMEM((CN_A,), jnp.float32),
        pltpu.VMEM((CN_A,), jnp.float32),
        pltpu.VMEM((CN_A,), jnp.int32),
        pltpu.VMEM((CN_A,), jnp.int32),
        pltpu.SemaphoreType.DMA,
        pltpu.SemaphoreType.DMA,
        pltpu.SemaphoreType.DMA,
        pltpu.SemaphoreType.DMA,
    ],
)
def _dist_switch(dist_hbm, idx_hbm, d_out, sw_out, m_out,
                 idx0, idx1, d0, d1, sw0, sw1, m0, m1,
                 sem0, sem1, osem0, osem1):
    wid, start_blk = _worker_start()
    idxs = (idx0, idx1)
    ds = (d0, d1)
    sws = (sw0, sw1)
    ms = (m0, m1)
    sems = (sem0, sem1)
    osems = (osem0, osem1)

    def stage_fire(j):
        p = j % 2
        base_i = (start_blk + j * CB_A) * 128
        pltpu.sync_copy(idx_hbm.at[pl.ds(base_i, CN_A)], idxs[p])
        return pltpu.async_copy(dist_hbm.at[idxs[p]], ds[p], sems[p])

    cps = [stage_fire(0), None]
    outs = [None, None]
    for j in range(NCH_A):
        p = j % 2
        q = (j + 1) % 2
        if j + 1 < NCH_A:
            if outs[q] is not None:
                for h in outs[q]:
                    h.wait()
                outs[q] = None
            cps[q] = stage_fire(j + 1)
        cps[p].wait()
        if outs[p] is not None:
            for h in outs[p]:
                h.wait()
            outs[p] = None

        def ew(i, carry2):
            _switch_step(ds[p], sws[p], ms[p], i)
            return carry2

        lax.fori_loop(0, CN_A // 16, ew, 0, unroll=4)
        base_i = (start_blk + j * CB_A) * 128
        outs[p] = (
            pltpu.async_copy(ds[p], d_out.at[pl.ds(base_i, CN_A)], osems[p]),
            pltpu.async_copy(sws[p], sw_out.at[pl.ds(base_i, CN_A)], osems[p]),
            pltpu.async_copy(ms[p], m_out.at[pl.ds(base_i, CN_A)], osems[p]),
        )

    for pair in outs:
        if pair is not None:
            for h in pair:
                h.wait()

    @pl.when(wid < EXTRA)
    def _():
        base_i = (start_blk + BASE_BLKS) * 128
        pltpu.sync_copy(idx_hbm.at[pl.ds(base_i, 128)], idx0.at[pl.ds(0, 128)])
        pltpu.async_copy(dist_hbm.at[idx0.at[pl.ds(0, 128)]],
                         d0.at[pl.ds(0, 128)], sem0).wait()

        def ew2(i, carry2):
            _switch_step(d0, sw0, m0, i)
            return carry2

        lax.fori_loop(0, 8, ew2, 0, unroll=4)
        pltpu.sync_copy(d0.at[pl.ds(0, 128)], d_out.at[pl.ds(base_i, 128)])
        pltpu.sync_copy(sw0.at[pl.ds(0, 128)], sw_out.at[pl.ds(base_i, 128)])
        pltpu.sync_copy(m0.at[pl.ds(0, 128)], m_out.at[pl.ds(base_i, 128)])


@functools.partial(
    pl.kernel,
    out_type=jax.ShapeDtypeStruct((3 * E_FILTER,), jnp.float32),
    mesh=plsc.VectorSubcoreMesh(core_axis_name="c", subcore_axis_name="s"),
    scratch_types=[
        pltpu.VMEM((CN_B,), jnp.int32),
        pltpu.VMEM((3 * CN_B,), jnp.int32),
        pltpu.VMEM((3 * CN_B,), jnp.int32),
        pltpu.VMEM((3 * CN_B,), jnp.float32),
        pltpu.VMEM((3 * CN_B,), jnp.float32),
        pltpu.SemaphoreType.DMA,
        pltpu.SemaphoreType.DMA,
        pltpu.SemaphoreType.DMA,
        pltpu.SemaphoreType.DMA,
    ],
)
def _vec_gather(vpf_hbm, idx_hbm, v_out,
                idx_v, idxf0, idxf1, v0, v1, sem0, sem1, osem0, osem1):
    wid, start_blk = _worker_start()
    idxfs = (idxf0, idxf1)
    vs = (v0, v1)
    sems = (sem0, sem1)
    osems = (osem0, osem1)

    def stage_expand(j, idxf_v):
        base_i = (start_blk + j * CB_B) * 128
        pltpu.sync_copy(idx_hbm.at[pl.ds(base_i, CN_B)], idx_v)

        def expand(g, carry2):
            _expand_step(idx_v, idxf_v, g)
            return carry2

        lax.fori_loop(0, CN_B // 16, expand, 0, unroll=4)

    stage_expand(0, idxf0)
    cps = [pltpu.async_copy(vpf_hbm.at[idxf0], v0, sem0), None]
    outs = [None, None]
    for j in range(NCH_B):
        p = j % 2
        q = (j + 1) % 2
        if j + 1 < NCH_B:
            stage_expand(j + 1, idxfs[q])
            if outs[q] is not None:
                outs[q].wait()
                outs[q] = None
            cps[q] = pltpu.async_copy(vpf_hbm.at[idxfs[q]], vs[q], sems[q])
        else:
            if outs[q] is not None:
                outs[q].wait()
                outs[q] = None

            @pl.when(wid < EXTRA)
            def _():
                base_i = (start_blk + BASE_BLKS) * 128
                pltpu.sync_copy(idx_hbm.at[pl.ds(base_i, 128)],
                                idx_v.at[pl.ds(0, 128)])

                def expand2(g, carry2):
                    _expand_step(idx_v, idxfs[q], g)
                    return carry2

                lax.fori_loop(0, 8, expand2, 0, unroll=4)
                pltpu.async_copy(vpf_hbm.at[idxfs[q].at[pl.ds(0, 384)]],
                                 vs[q].at[pl.ds(0, 384)], sems[q]).wait()
                pltpu.sync_copy(vs[q].at[pl.ds(0, 384)],
                                v_out.at[pl.ds(base_i * 3, 384)])
        base_i = (start_blk + j * CB_B) * 128
        cps[p].wait()
        outs[p] = pltpu.async_copy(vs[p], v_out.at[pl.ds(base_i * 3, 3 * CN_B)],
                                   osems[p])

    for h in outs:
        if h is not None:
            h.wait()


def kernel(vec, distances, filter_indices):
    d, sw, m = _dist_switch(distances, filter_indices)
    vpf = vec.T.reshape(3, E_PARENT // 128, 128).transpose(1, 0, 2).reshape(-1)
    vflat = _vec_gather(vpf, filter_indices)
    v = vflat.reshape(E_FILTER // 128, 3, 128).transpose(1, 0, 2)
    v = v.reshape(3, E_FILTER).T
    return v, d, sw, m.astype(jnp.bool_)


# speedup vs baseline: 25.6510x; 1.0009x over previous
import functools

import jax
import jax.numpy as jnp
from jax import lax
from jax.experimental import pallas as pl
from jax.experimental.pallas import tpu as pltpu
from jax.experimental.pallas import tpu_sc as plsc

CUTOFF = 0.5
E_PARENT = 6400000
E_FILTER = 1600000

NW = 32
NBLK = E_FILTER // 128
BASE_BLKS = NBLK // NW
EXTRA = NBLK - BASE_BLKS * NW

CB_A = 78
CN_A = CB_A * 128
NCH_A = BASE_BLKS // CB_A

CB_B = 65
CN_B = CB_B * 128
NCH_B = BASE_BLKS // CB_B

_PI = 3.14159265358979
_C3 = -1.0 / 6.0
_C5 = 1.0 / 120.0
_C7 = -1.0 / 5040.0
_C9 = 1.0 / 362880.0


def _expand_step(idx_v, idxf_v, g):
    a = idx_v[pl.ds(g * 16, 16)]
    p0 = (a >> 7) * 384 + (a & 127)
    b = g >> 3
    r = g & 7
    dst = b * 384 + r * 16
    idxf_v[pl.ds(dst, 16)] = p0
    idxf_v[pl.ds(dst + 128, 16)] = p0 + 128
    idxf_v[pl.ds(dst + 256, 16)] = p0 + 256


def _switch_step(d_v, sw_v, m_v, i):
    d16 = d_v[pl.ds(i * 16, 16)]
    s = (d16 * (2.0 * _PI)) - (0.5 * _PI)
    s2 = s * s
    p = s2 * _C9 + _C7
    p = s2 * p + _C5
    p = s2 * p + _C3
    sin_s = s + s * (s2 * p)
    mask = d16 < CUTOFF
    sw_v[pl.ds(i * 16, 16)] = jnp.where(mask, 0.5 - 0.5 * sin_s,
                                        jnp.zeros((16,), jnp.float32))
    m_v[pl.ds(i * 16, 16)] = jnp.where(mask, jnp.ones((16,), jnp.int32),
                                       jnp.zeros((16,), jnp.int32))


def _worker_start(_=None):
    wid = lax.axis_index("s") * 2 + lax.axis_index("c")
    return wid, wid * BASE_BLKS + jnp.minimum(wid, EXTRA)


@functools.partial(
    pl.kernel,
    out_type=[
        jax.ShapeDtypeStruct((E_FILTER,), jnp.float32),
        jax.ShapeDtypeStruct((E_FILTER,), jnp.float32),
        jax.ShapeDtypeStruct((E_FILTER,), jnp.int32),
    ],
    mesh=plsc.VectorSubcoreMesh(core_axis_name="c", subcore_axis_name="s"),
    scratch_types=[
        pltpu.VMEM((CN_A,), jnp.int32),
        pltpu.VMEM((CN_A,), jnp.int32),
        pltpu.VMEM((CN_A,), jnp.float32),
        pltpu.VMEM((CN_A,), jnp.float32),
        pltpu.VMEM((CN_A,), jnp.float32),
        pltpu.VMEM((CN_A,), jnp.float32),
        pltpu.VMEM((CN_A,), jnp.int32),
        pltpu.VMEM((CN_A,), jnp.int32),
        pltpu.SemaphoreType.DMA,
        pltpu.SemaphoreType.DMA,
        pltpu.SemaphoreType.DMA,
        pltpu.SemaphoreType.DMA,
    ],
)
def _dist_switch(dist_hbm, idx_hbm, d_out, sw_out, m_out,
                 idx0, idx1, d0, d1, sw0, sw1, m0, m1,
                 sem0, sem1, osem0, osem1):
    wid, start_blk = _worker_start()
    idxs = (idx0, idx1)
    ds = (d0, d1)
    sws = (sw0, sw1)
    ms = (m0, m1)
    sems = (sem0, sem1)
    osems = (osem0, osem1)

    def stage_fire(j):
        p = j % 2
        base_i = (start_blk + j * CB_A) * 128
        pltpu.sync_copy(idx_hbm.at[pl.ds(base_i, CN_A)], idxs[p])
        return pltpu.async_copy(dist_hbm.at[idxs[p]], ds[p], sems[p])

    cps = [stage_fire(0), None]
    outs = [None, None]
    for j in range(NCH_A):
        p = j % 2
        q = (j + 1) % 2
        if j + 1 < NCH_A:
            if outs[q] is not None:
                for h in outs[q]:
                    h.wait()
                outs[q] = None
            cps[q] = stage_fire(j + 1)
        cps[p].wait()
        if outs[p] is not None:
            for h in outs[p]:
                h.wait()
            outs[p] = None

        def ew(i, carry2):
            _switch_step(ds[p], sws[p], ms[p], i)
            return carry2

        lax.fori_loop(0, CN_A // 16, ew, 0, unroll=4)
        base_i = (start_blk + j * CB_A) * 128
        outs[p] = (
            pltpu.async_copy(ds[p], d_out.at[pl.ds(base_i, CN_A)], osems[p]),
            pltpu.async_copy(sws[p], sw_out.at[pl.ds(base_i, CN_A)], osems[p]),
            pltpu.async_copy(ms[p], m_out.at[pl.ds(base_i, CN_A)], osems[p]),
        )

    for pair in outs:
        if pair is not None:
            for h in pair:
                h.wait()

    @pl.when(wid < EXTRA)
    def _():
        base_i = (start_blk + BASE_BLKS) * 128
        pltpu.sync_copy(idx_hbm.at[pl.ds(base_i, 128)], idx0.at[pl.ds(0, 128)])
        pltpu.async_copy(dist_hbm.at[idx0.at[pl.ds(0, 128)]],
                         d0.at[pl.ds(0, 128)], sem0).wait()

        def ew2(i, carry2):
            _switch_step(d0, sw0, m0, i)
            return carry2

        lax.fori_loop(0, 8, ew2, 0, unroll=4)
        pltpu.sync_copy(d0.at[pl.ds(0, 128)], d_out.at[pl.ds(base_i, 128)])
        pltpu.sync_copy(sw0.at[pl.ds(0, 128)], sw_out.at[pl.ds(base_i, 128)])
        pltpu.sync_copy(m0.at[pl.ds(0, 128)], m_out.at[pl.ds(base_i, 128)])


@functools.partial(
    pl.kernel,
    out_type=jax.ShapeDtypeStruct((3 * E_FILTER,), jnp.float32),
    mesh=plsc.VectorSubcoreMesh(core_axis_name="c", subcore_axis_name="s"),
    scratch_types=[
        pltpu.VMEM((CN_B,), jnp.int32),
        pltpu.VMEM((3 * CN_B,), jnp.int32),
        pltpu.VMEM((3 * CN_B,), jnp.int32),
        pltpu.VMEM((3 * CN_B,), jnp.float32),
        pltpu.VMEM((3 * CN_B,), jnp.float32),
        pltpu.SemaphoreType.DMA,
        pltpu.SemaphoreType.DMA,
        pltpu.SemaphoreType.DMA,
        pltpu.SemaphoreType.DMA,
    ],
)
def _vec_gather(vpf_hbm, idx_hbm, v_out,
                idx_v, idxf0, idxf1, v0, v1, sem0, sem1, osem0, osem1):
    wid, start_blk = _worker_start()
    idxfs = (idxf0, idxf1)
    vs = (v0, v1)
    sems = (sem0, sem1)
    osems = (osem0, osem1)

    def stage_expand(j, idxf_v):
        base_i = (start_blk + j * CB_B) * 128
        pltpu.sync_copy(idx_hbm.at[pl.ds(base_i, CN_B)], idx_v)

        def expand(g, carry2):
            _expand_step(idx_v, idxf_v, g)
            return carry2

        lax.fori_loop(0, CN_B // 16, expand, 0, unroll=4)

    stage_expand(0, idxf0)
    cps = [pltpu.async_copy(vpf_hbm.at[idxf0], v0, sem0), None]
    outs = [None, None]
    for j in range(NCH_B):
        p = j % 2
        q = (j + 1) % 2
        if j + 1 < NCH_B:
            stage_expand(j + 1, idxfs[q])
            if outs[q] is not None:
                outs[q].wait()
                outs[q] = None
            cps[q] = pltpu.async_copy(vpf_hbm.at[idxfs[q]], vs[q], sems[q])
        else:
            if outs[q] is not None:
                outs[q].wait()
                outs[q] = None

            @pl.when(wid < EXTRA)
            def _():
                base_i = (start_blk + BASE_BLKS) * 128
                pltpu.sync_copy(idx_hbm.at[pl.ds(base_i, 128)],
                                idx_v.at[pl.ds(0, 128)])

                def expand2(g, carry2):
                    _expand_step(idx_v, idxfs[q], g)
                    return carry2

                lax.fori_loop(0, 8, expand2, 0, unroll=4)
                pltpu.async_copy(vpf_hbm.at[idxfs[q].at[pl.ds(0, 384)]],
                                 vs[q].at[pl.ds(0, 384)], sems[q]).wait()
                pltpu.sync_copy(vs[q].at[pl.ds(0, 384)],
                                v_out.at[pl.ds(base_i * 3, 384)])
        base_i = (start_blk + j * CB_B) * 128
        cps[p].wait()
        outs[p] = pltpu.async_copy(vs[p], v_out.at[pl.ds(base_i * 3, 3 * CN_B)],
                                   osems[p])

    for h in outs:
        if h is not None:
            h.wait()


def kernel(vec, distances, filter_indices):
    vpf = vec.T.reshape(3, E_PARENT // 128, 128).transpose(1, 0, 2).reshape(-1)
    vflat = _vec_gather(vpf, filter_indices)
    d, sw, m = _dist_switch(distances, filter_indices)
    v = vflat.reshape(E_FILTER // 128, 3, 128).transpose(1, 0, 2)
    v = v.reshape(3, E_FILTER).T
    return v, d, sw, m.astype(jnp.bool_)
